# Initial kernel scaffold; baseline (speedup 1.0000x reference)
import jax
import jax.numpy as jnp
from jax.experimental import pallas as pl


def kernel(h, edge_index, fwd_msg_W, fwd_msg_b, fwd_Wih, fwd_Whh, fwd_bih, fwd_bhh, bwd_msg_W, bwd_msg_b, bwd_Wih, bwd_Whh, bwd_bih, bwd_bhh):
    raise NotImplementedError("write your pallas kernel here")


# speedup vs baseline: 10.0824x; 10.0824x over previous
import functools

import jax
import jax.numpy as jnp
from jax import lax
from jax.experimental import pallas as pl
from jax.experimental.pallas import tpu as pltpu
from jax.experimental.pallas import tpu_sc as plsc

NDIM = 128
HID = 64
N_NODES = 10000
N_EDGES = 320000

_NC, _NS, _LANES = 2, 16, 16
_NW = _NC * _NS
_N_PAD = 10240
_ROWS_PER_TILE = _N_PAD // _NS
_CHUNK = 128
_E_PAD = 327680
_EDGES_PER_W = _E_PAD // _NW
_CHUNKS_PER_W = _EDGES_PER_W // _CHUNK


def _sc_segment_sums(hf, hb, src, dst, z2, z1):
    mesh = plsc.VectorSubcoreMesh(
        core_axis_name="c", subcore_axis_name="s",
        num_cores=_NC, num_subcores=_NS)

    @functools.partial(
        pl.kernel,
        out_type=(
            jax.ShapeDtypeStruct((_NC * _N_PAD, HID), jnp.float32),
            jax.ShapeDtypeStruct((_NC * _N_PAD, HID), jnp.float32),
            jax.ShapeDtypeStruct((_NC * _N_PAD,), jnp.float32),
            jax.ShapeDtypeStruct((_NC * _N_PAD,), jnp.float32),
        ),
        mesh=mesh,
        compiler_params=pltpu.CompilerParams(use_tc_tiling_on_sc=False),
        scratch_types=[
            pltpu.VMEM_SHARED((_N_PAD, HID), jnp.float32),
            pltpu.VMEM_SHARED((_N_PAD, HID), jnp.float32),
            pltpu.VMEM_SHARED((_N_PAD,), jnp.float32),
            pltpu.VMEM_SHARED((_N_PAD,), jnp.float32),
            pltpu.VMEM((_CHUNK,), jnp.int32),
            pltpu.VMEM((_CHUNK,), jnp.int32),
            pltpu.VMEM((_CHUNK, HID), jnp.float32),
            pltpu.VMEM((_CHUNK, HID), jnp.float32),
            pltpu.VMEM((_CHUNK,), jnp.float32),
            pltpu.SemaphoreType.DMA,
            pltpu.SemaphoreType.DMA,
        ],
    )
    def k(hf_hbm, hb_hbm, src_hbm, dst_hbm, z2_hbm, z1_hbm,
          sf_out, sb_out, degf_out, degb_out,
          sf_sh, sb_sh, degf_sh, degb_sh,
          src_v, dst_v, rows_f, rows_b, ones_v, sem_a, sem_b):
        c = lax.axis_index("c")
        s = lax.axis_index("s")
        wid = s * _NC + c
        r0 = s * _ROWS_PER_TILE
        pltpu.sync_copy(z2_hbm, sf_sh.at[pl.ds(r0, _ROWS_PER_TILE)])
        pltpu.sync_copy(z2_hbm, sb_sh.at[pl.ds(r0, _ROWS_PER_TILE)])
        pltpu.sync_copy(z1_hbm, degf_sh.at[pl.ds(r0, _ROWS_PER_TILE)])
        pltpu.sync_copy(z1_hbm, degb_sh.at[pl.ds(r0, _ROWS_PER_TILE)])
        for g in range(_CHUNK // _LANES):
            ones_v[pl.ds(g * _LANES, _LANES)] = jnp.full((_LANES,), 1.0, jnp.float32)
        plsc.subcore_barrier()

        base = wid * jnp.int32(_EDGES_PER_W)

        def body(i, carry):
            off = pl.multiple_of(base + i * jnp.int32(_CHUNK), _CHUNK)
            pltpu.sync_copy(src_hbm.at[pl.ds(off, _CHUNK)], src_v)
            pltpu.sync_copy(dst_hbm.at[pl.ds(off, _CHUNK)], dst_v)
            ga = pltpu.async_copy(hf_hbm.at[src_v], rows_f, sem_a)
            gb = pltpu.async_copy(hb_hbm.at[dst_v], rows_b, sem_b)
            ga.wait()
            gb.wait()
            pltpu.sync_copy(rows_f, sf_sh.at[dst_v], add=True)
            pltpu.sync_copy(rows_b, sb_sh.at[src_v], add=True)
            pltpu.sync_copy(ones_v, degf_sh.at[dst_v], add=True)
            pltpu.sync_copy(ones_v, degb_sh.at[src_v], add=True)
            return carry

        lax.fori_loop(jnp.int32(0), jnp.int32(_CHUNKS_PER_W), body, jnp.int32(0))
        plsc.subcore_barrier()

        out0 = pl.multiple_of(c * _N_PAD + r0, _ROWS_PER_TILE)
        pltpu.sync_copy(sf_sh.at[pl.ds(r0, _ROWS_PER_TILE)], sf_out.at[pl.ds(out0, _ROWS_PER_TILE)])
        pltpu.sync_copy(sb_sh.at[pl.ds(r0, _ROWS_PER_TILE)], sb_out.at[pl.ds(out0, _ROWS_PER_TILE)])
        pltpu.sync_copy(degf_sh.at[pl.ds(r0, _ROWS_PER_TILE)], degf_out.at[pl.ds(out0, _ROWS_PER_TILE)])
        pltpu.sync_copy(degb_sh.at[pl.ds(r0, _ROWS_PER_TILE)], degb_out.at[pl.ds(out0, _ROWS_PER_TILE)])

    return k(hf, hb, src, dst, z2, z1)


_TC_ROWS = 2048
_TC_GRID = (_N_PAD // _TC_ROWS,)

_DN = (((1,), (1,)), ((), ()))
_PREC = lax.Precision.DEFAULT


def _tc_body(sf_ref, sb_ref, hf_ref, hb_ref, degf_ref, degb_ref,
             fW1, fW2, fmb, fWih, fWhh, fbih, fbhh,
             bW1, bW2, bmb, bWih, bWhh, bbih, bbhh,
             of_ref, ob_ref):
    def one(s2, h, dg, W1, W2, mb, Wih, Whh, bih, bhh, out):
        S = s2[0] + s2[1]
        hh = h[...]
        msg_self = lax.dot_general(hh, W2[...], _DN, precision=_PREC) + mb[...]
        aggr = lax.dot_general(S, W1[...], _DN, precision=_PREC) + dg[...] * msg_self

        def mm(x, W, k):
            return lax.dot_general(x, W[k * HID:(k + 1) * HID], _DN, precision=_PREC)

        r = jax.nn.sigmoid(mm(aggr, Wih, 0) + bih[0:1] + mm(hh, Whh, 0) + bhh[0:1])
        z = jax.nn.sigmoid(mm(aggr, Wih, 1) + bih[1:2] + mm(hh, Whh, 1) + bhh[1:2])
        n = jnp.tanh(mm(aggr, Wih, 2) + bih[2:3] + r * (mm(hh, Whh, 2) + bhh[2:3]))
        out[...] = (1.0 - z) * n + z * hh

    one(sf_ref, hf_ref, degf_ref, fW1, fW2, fmb, fWih, fWhh, fbih, fbhh, of_ref)
    one(sb_ref, hb_ref, degb_ref, bW1, bW2, bmb, bWih, bWhh, bbih, bbhh, ob_ref)


def _rows(i):
    return (i, i * 0)


def _rows3(i):
    return (i * 0, i, i * 0)


def _fixed(i):
    return (i * 0, i * 0)


_W_SHAPES = [(NDIM, HID), (NDIM, HID), (1, NDIM), (3 * HID, NDIM), (3 * HID, HID), (3, HID), (3, HID)]

_TC_IN_SPECS = (
    [pl.BlockSpec((_NC, _TC_ROWS, HID), _rows3)] * 2
    + [pl.BlockSpec((_TC_ROWS, HID), _rows)] * 2
    + [pl.BlockSpec((_TC_ROWS, 1), _rows)] * 2
    + [pl.BlockSpec(shp, _fixed) for shp in _W_SHAPES] * 2
)
_TC_OUT_SPECS = [pl.BlockSpec((_TC_ROWS, HID), _rows)] * 2
_TC_OUT_SHAPE = [jax.ShapeDtypeStruct((_N_PAD, HID), jnp.float32)] * 2


def _tc_layer(sf, sb, hf, hb, degf, degb, wf, wb):
    return pl.pallas_call(
        _tc_body,
        grid=_TC_GRID,
        in_specs=_TC_IN_SPECS,
        out_specs=_TC_OUT_SPECS,
        out_shape=_TC_OUT_SHAPE,
    )(sf, sb, hf, hb, degf, degb, *wf, *wb)


def _prep_weights(msg_W, msg_b, Wih, Whh, bih, bhh):
    return (msg_W[:, :HID], msg_W[:, HID:], msg_b.reshape(1, NDIM),
            Wih, Whh, bih.reshape(3, HID), bhh.reshape(3, HID))


def kernel(h, edge_index, fwd_msg_W, fwd_msg_b, fwd_Wih, fwd_Whh, fwd_bih, fwd_bhh,
           bwd_msg_W, bwd_msg_b, bwd_Wih, bwd_Whh, bwd_bih, bwd_bhh):
    h = h.astype(jnp.float32)
    src = edge_index[0].astype(jnp.int32)
    dst = edge_index[1].astype(jnp.int32)
    pad_idx = (jnp.arange(_E_PAD - N_EDGES, dtype=jnp.int32) % (_N_PAD - N_NODES)) + N_NODES
    srcp = jnp.concatenate([src, pad_idx])
    dstp = jnp.concatenate([dst, pad_idx])
    hf = jnp.pad(h[:, :HID], ((0, _N_PAD - N_NODES), (0, 0)))
    hb = jnp.pad(h[:, HID:], ((0, _N_PAD - N_NODES), (0, 0)))
    z2 = jnp.zeros((_ROWS_PER_TILE, HID), jnp.float32)
    z1 = jnp.zeros((_ROWS_PER_TILE,), jnp.float32)

    wf = [_prep_weights(fwd_msg_W[l], fwd_msg_b[l], fwd_Wih[l], fwd_Whh[l], fwd_bih[l], fwd_bhh[l])
          for l in range(2)]
    wb = [_prep_weights(bwd_msg_W[l], bwd_msg_b[l], bwd_Wih[l], bwd_Whh[l], bwd_bih[l], bwd_bhh[l])
          for l in range(2)]

    degf_col = degb_col = None
    for l in range(2):
        sf_flat, sb_flat, degf_flat, degb_flat = _sc_segment_sums(hf, hb, srcp, dstp, z2, z1)
        sf = sf_flat.reshape(_NC, _N_PAD, HID)
        sb = sb_flat.reshape(_NC, _N_PAD, HID)
        if l == 0:
            degf_col = (degf_flat[:_N_PAD] + degf_flat[_N_PAD:]).reshape(_N_PAD, 1)
            degb_col = (degb_flat[:_N_PAD] + degb_flat[_N_PAD:]).reshape(_N_PAD, 1)
        hf, hb = _tc_layer(sf, sb, hf, hb, degf_col, degb_col, wf[l], wb[l])

    return jnp.concatenate([hf[:N_NODES], hb[:N_NODES]], axis=1)


# speedup vs baseline: 17.9499x; 1.7803x over previous
import functools

import jax
import jax.numpy as jnp
from jax import lax
from jax.experimental import pallas as pl
from jax.experimental.pallas import tpu as pltpu
from jax.experimental.pallas import tpu_sc as plsc

NDIM = 128
HID = 64
N_NODES = 10000
N_EDGES = 320000

_NC, _NS, _LANES = 2, 16, 16
_N_PAD = 10240
_ROWS_PER_TILE = _N_PAD // _NS
_CHUNK = 128
_E_PAD = 327680
_E_ROWS = _E_PAD // _CHUNK
_CHUNKS_PER_T = _E_ROWS // _NS
_EPT_HALF = _CHUNKS_PER_T // 2


def _sc_segment_sums(htab, gidx, sidx, z2, z1, with_deg):
    mesh = plsc.VectorSubcoreMesh(
        core_axis_name="c", subcore_axis_name="s",
        num_cores=_NC, num_subcores=_NS)

    out_type = [jax.ShapeDtypeStruct((_NC * _N_PAD, HID), jnp.float32)]
    scratch = [
        pltpu.VMEM_SHARED((_N_PAD, HID), jnp.float32),
        pltpu.VMEM((_CHUNKS_PER_T, _CHUNK), jnp.int32),
        pltpu.VMEM((_CHUNKS_PER_T, _CHUNK), jnp.int32),
        pltpu.VMEM((2, _CHUNK, HID), jnp.float32),
        pltpu.SemaphoreType.DMA,
        pltpu.SemaphoreType.DMA,
    ]
    if with_deg:
        out_type += [jax.ShapeDtypeStruct((_NC * _N_PAD,), jnp.float32)]
        scratch += [
            pltpu.VMEM_SHARED((_N_PAD,), jnp.float32),
            pltpu.VMEM((_CHUNK,), jnp.float32),
        ]

    @functools.partial(
        pl.kernel,
        out_type=tuple(out_type),
        mesh=mesh,
        compiler_params=pltpu.CompilerParams(use_tc_tiling_on_sc=False),
        scratch_types=scratch,
    )
    def k(htab_hbm, gidx_hbm, sidx_hbm, z2_hbm, z1_hbm, *rest):
        if with_deg:
            (s_out, deg_out, acc_sh, gidx_v, sidx_v, rows, sem0, sem1,
             deg_sh, ones_v) = rest
        else:
            s_out, acc_sh, gidx_v, sidx_v, rows, sem0, sem1 = rest
        gsem = (sem0, sem1)
        c = lax.axis_index("c")
        s = lax.axis_index("s")
        r0 = s * _ROWS_PER_TILE
        pltpu.sync_copy(z2_hbm, acc_sh.at[pl.ds(r0, _ROWS_PER_TILE)])
        if with_deg:
            pltpu.sync_copy(z1_hbm, deg_sh.at[pl.ds(r0, _ROWS_PER_TILE)])
            for g in range(_CHUNK // _LANES):
                ones_v[pl.ds(g * _LANES, _LANES)] = jnp.full((_LANES,), 1.0, jnp.float32)
        slab0 = pl.multiple_of(c * jnp.int32(_E_ROWS) + s * jnp.int32(_CHUNKS_PER_T),
                               _CHUNKS_PER_T)
        pltpu.sync_copy(gidx_hbm.at[pl.ds(slab0, _CHUNKS_PER_T)], gidx_v)
        pltpu.sync_copy(sidx_hbm.at[pl.ds(slab0, _CHUNKS_PER_T)], sidx_v)
        plsc.subcore_barrier()

        last = jnp.int32(_CHUNKS_PER_T - 1)

        def issue_gather(i, b):
            row = jnp.minimum(i, last)
            pltpu.async_copy(htab_hbm.at[gidx_v.at[row]], rows.at[jnp.int32(b)], gsem[b])

        def wait_gather(b):
            pltpu.make_async_copy(
                htab_hbm.at[gidx_v.at[jnp.int32(0)]], rows.at[jnp.int32(b)], gsem[b]).wait()

        issue_gather(jnp.int32(0), 0)
        issue_gather(jnp.int32(1), 1)

        def body(j, carry):
            for b in range(2):
                i = j * jnp.int32(2) + jnp.int32(b)
                wait_gather(b)
                pltpu.sync_copy(rows.at[jnp.int32(b)], acc_sh.at[sidx_v.at[i]], add=True)
                if with_deg:
                    pltpu.sync_copy(ones_v, deg_sh.at[sidx_v.at[i]], add=True)
                issue_gather(i + jnp.int32(2), b)
            return carry

        lax.fori_loop(jnp.int32(0), jnp.int32(_EPT_HALF), body, jnp.int32(0))
        wait_gather(0)
        wait_gather(1)
        plsc.subcore_barrier()

        out0 = pl.multiple_of(c * jnp.int32(_N_PAD) + r0, _ROWS_PER_TILE)
        pltpu.sync_copy(acc_sh.at[pl.ds(r0, _ROWS_PER_TILE)], s_out.at[pl.ds(out0, _ROWS_PER_TILE)])
        if with_deg:
            pltpu.sync_copy(deg_sh.at[pl.ds(r0, _ROWS_PER_TILE)], deg_out.at[pl.ds(out0, _ROWS_PER_TILE)])

    return k(htab, gidx, sidx, z2, z1)


_TC_ROWS = 2048
_DIRBLKS = _N_PAD // _TC_ROWS
_TC_GRID = (_NC * _DIRBLKS,)

_DN = (((1,), (1,)), ((), ()))
_PREC = lax.Precision.DEFAULT


def _tc_body(s_ref, h_ref, deg_ref, W1, W2, mb, Wih, Whh, bih, bhh, out_ref):
    S = s_ref[...]
    hh = h_ref[...]
    dg = deg_ref[...]
    msg_self = lax.dot_general(hh, W2[0], _DN, precision=_PREC) + mb[0]
    aggr = lax.dot_general(S, W1[0], _DN, precision=_PREC) + dg * msg_self

    def mm(x, W, k):
        return lax.dot_general(x, W[0, k * HID:(k + 1) * HID], _DN, precision=_PREC)

    r = jax.nn.sigmoid(mm(aggr, Wih, 0) + bih[0, 0:1] + mm(hh, Whh, 0) + bhh[0, 0:1])
    z = jax.nn.sigmoid(mm(aggr, Wih, 1) + bih[0, 1:2] + mm(hh, Whh, 1) + bhh[0, 1:2])
    n = jnp.tanh(mm(aggr, Wih, 2) + bih[0, 2:3] + r * (mm(hh, Whh, 2) + bhh[0, 2:3]))
    out_ref[...] = (1.0 - z) * n + z * hh


def _rows(i):
    return (i, i * 0)


def _dirw(i):
    return (i // _DIRBLKS, i * 0, i * 0)


_W_SHAPES = [(NDIM, HID), (NDIM, HID), (1, NDIM), (3 * HID, NDIM), (3 * HID, HID), (3, HID), (3, HID)]

_TC_IN_SPECS = (
    [pl.BlockSpec((_TC_ROWS, HID), _rows),
     pl.BlockSpec((_TC_ROWS, HID), _rows),
     pl.BlockSpec((_TC_ROWS, 1), _rows)]
    + [pl.BlockSpec((1,) + shp, _dirw) for shp in _W_SHAPES]
)
_TC_OUT_SPECS = pl.BlockSpec((_TC_ROWS, HID), _rows)
_TC_OUT_SHAPE = jax.ShapeDtypeStruct((_NC * _N_PAD, HID), jnp.float32)


def _tc_layer(s, htab, deg, ws):
    return pl.pallas_call(
        _tc_body,
        grid=_TC_GRID,
        in_specs=_TC_IN_SPECS,
        out_specs=_TC_OUT_SPECS,
        out_shape=_TC_OUT_SHAPE,
    )(s, htab, deg, *ws)


def _prep_weights(fw, bw):
    def stack(f, b):
        return jnp.stack([f, b])
    f_msg_W, f_msg_b, f_Wih, f_Whh, f_bih, f_bhh = fw
    b_msg_W, b_msg_b, b_Wih, b_Whh, b_bih, b_bhh = bw
    return (
        stack(f_msg_W[:, :HID], b_msg_W[:, :HID]),
        stack(f_msg_W[:, HID:], b_msg_W[:, HID:]),
        stack(f_msg_b.reshape(1, NDIM), b_msg_b.reshape(1, NDIM)),
        stack(f_Wih, b_Wih),
        stack(f_Whh, b_Whh),
        stack(f_bih.reshape(3, HID), b_bih.reshape(3, HID)),
        stack(f_bhh.reshape(3, HID), b_bhh.reshape(3, HID)),
    )


def kernel(h, edge_index, fwd_msg_W, fwd_msg_b, fwd_Wih, fwd_Whh, fwd_bih, fwd_bhh,
           bwd_msg_W, bwd_msg_b, bwd_Wih, bwd_Whh, bwd_bih, bwd_bhh):
    h = h.astype(jnp.float32)
    src = edge_index[0].astype(jnp.int32)
    dst = edge_index[1].astype(jnp.int32)
    pad_idx = (jnp.arange(_E_PAD - N_EDGES, dtype=jnp.int32) % (_N_PAD - N_NODES)) + N_NODES
    srcp = jnp.concatenate([src, pad_idx]).reshape(_E_ROWS, _CHUNK)
    dstp = jnp.concatenate([dst, pad_idx]).reshape(_E_ROWS, _CHUNK)
    gidx = jnp.concatenate([srcp, dstp + jnp.int32(_N_PAD)], axis=0)
    sidx = jnp.concatenate([dstp, srcp], axis=0)
    htab = jnp.concatenate([
        jnp.pad(h[:, :HID], ((0, _N_PAD - N_NODES), (0, 0))),
        jnp.pad(h[:, HID:], ((0, _N_PAD - N_NODES), (0, 0))),
    ], axis=0)
    z2 = jnp.zeros((_ROWS_PER_TILE, HID), jnp.float32)
    z1 = jnp.zeros((_ROWS_PER_TILE,), jnp.float32)

    ws = [_prep_weights((fwd_msg_W[l], fwd_msg_b[l], fwd_Wih[l], fwd_Whh[l], fwd_bih[l], fwd_bhh[l]),
                        (bwd_msg_W[l], bwd_msg_b[l], bwd_Wih[l], bwd_Whh[l], bwd_bih[l], bwd_bhh[l]))
          for l in range(2)]

    deg_col = None
    for l in range(2):
        res = _sc_segment_sums(htab, gidx, sidx, z2, z1, with_deg=(l == 0))
        if l == 0:
            deg_col = res[1].reshape(_NC * _N_PAD, 1)
        htab = _tc_layer(res[0], htab, deg_col, ws[l])

    return jnp.concatenate([htab[:N_NODES], htab[_N_PAD:_N_PAD + N_NODES]], axis=1)


# speedup vs baseline: 18.0533x; 1.0058x over previous
import functools

import jax
import jax.numpy as jnp
from jax import lax
from jax.experimental import pallas as pl
from jax.experimental.pallas import tpu as pltpu
from jax.experimental.pallas import tpu_sc as plsc

NDIM = 128
HID = 64
N_NODES = 10000
N_EDGES = 320000

_NC, _NS, _LANES = 2, 16, 16
_N_PAD = 10240
_ROWS_PER_TILE = _N_PAD // _NS
_CHUNK = 128
_E_PAD = 327680
_E_ROWS = _E_PAD // _CHUNK
_CHUNKS_PER_T = _E_ROWS // _NS
_EPT_HALF = _CHUNKS_PER_T // 2


def _sc_segment_sums(htab, gidx, sidx, z2, z1, with_deg):
    mesh = plsc.VectorSubcoreMesh(
        core_axis_name="c", subcore_axis_name="s",
        num_cores=_NC, num_subcores=_NS)

    out_type = [jax.ShapeDtypeStruct((_NC * _N_PAD, HID), jnp.float32)]
    scratch = [
        pltpu.VMEM_SHARED((_N_PAD, HID), jnp.float32),
        pltpu.VMEM((_CHUNKS_PER_T, _CHUNK), jnp.int32),
        pltpu.VMEM((_CHUNKS_PER_T, _CHUNK), jnp.int32),
        pltpu.VMEM((2, _CHUNK, HID), jnp.float32),
        pltpu.SemaphoreType.DMA,
        pltpu.SemaphoreType.DMA,
    ]
    if with_deg:
        out_type += [jax.ShapeDtypeStruct((_NC * _N_PAD,), jnp.float32)]
        scratch += [
            pltpu.VMEM_SHARED((_N_PAD,), jnp.float32),
            pltpu.VMEM((_CHUNK,), jnp.float32),
        ]

    @functools.partial(
        pl.kernel,
        out_type=tuple(out_type),
        mesh=mesh,
        compiler_params=pltpu.CompilerParams(use_tc_tiling_on_sc=False),
        scratch_types=scratch,
    )
    def k(htab_hbm, gidx_hbm, sidx_hbm, z2_hbm, z1_hbm, *rest):
        if with_deg:
            (s_out, deg_out, acc_sh, gidx_v, sidx_v, rows, sem0, sem1,
             deg_sh, ones_v) = rest
        else:
            s_out, acc_sh, gidx_v, sidx_v, rows, sem0, sem1 = rest
        gsem = (sem0, sem1)
        c = lax.axis_index("c")
        s = lax.axis_index("s")
        r0 = s * _ROWS_PER_TILE
        pltpu.sync_copy(z2_hbm, acc_sh.at[pl.ds(r0, _ROWS_PER_TILE)])
        if with_deg:
            pltpu.sync_copy(z1_hbm, deg_sh.at[pl.ds(r0, _ROWS_PER_TILE)])
            for g in range(_CHUNK // _LANES):
                ones_v[pl.ds(g * _LANES, _LANES)] = jnp.full((_LANES,), 1.0, jnp.float32)
        slab0 = pl.multiple_of(c * jnp.int32(_E_ROWS) + s * jnp.int32(_CHUNKS_PER_T),
                               _CHUNKS_PER_T)
        pltpu.sync_copy(gidx_hbm.at[pl.ds(slab0, _CHUNKS_PER_T)], gidx_v)
        pltpu.sync_copy(sidx_hbm.at[pl.ds(slab0, _CHUNKS_PER_T)], sidx_v)
        plsc.subcore_barrier()

        last = jnp.int32(_CHUNKS_PER_T - 1)

        def issue_gather(i, b):
            row = jnp.minimum(i, last)
            pltpu.async_copy(htab_hbm.at[gidx_v.at[row]], rows.at[jnp.int32(b)], gsem[b])

        def wait_gather(b):
            pltpu.make_async_copy(
                htab_hbm.at[gidx_v.at[jnp.int32(0)]], rows.at[jnp.int32(b)], gsem[b]).wait()

        issue_gather(jnp.int32(0), 0)
        issue_gather(jnp.int32(1), 1)

        def body(j, carry):
            for b in range(2):
                i = j * jnp.int32(2) + jnp.int32(b)
                wait_gather(b)
                pltpu.sync_copy(rows.at[jnp.int32(b)], acc_sh.at[sidx_v.at[i]], add=True)
                if with_deg:
                    pltpu.sync_copy(ones_v, deg_sh.at[sidx_v.at[i]], add=True)
                issue_gather(i + jnp.int32(2), b)
            return carry

        lax.fori_loop(jnp.int32(0), jnp.int32(_EPT_HALF), body, jnp.int32(0))
        wait_gather(0)
        wait_gather(1)
        plsc.subcore_barrier()

        out0 = pl.multiple_of(c * jnp.int32(_N_PAD) + r0, _ROWS_PER_TILE)
        pltpu.sync_copy(acc_sh.at[pl.ds(r0, _ROWS_PER_TILE)], s_out.at[pl.ds(out0, _ROWS_PER_TILE)])
        if with_deg:
            pltpu.sync_copy(deg_sh.at[pl.ds(r0, _ROWS_PER_TILE)], deg_out.at[pl.ds(out0, _ROWS_PER_TILE)])

    return k(htab, gidx, sidx, z2, z1)


_TC_ROWS = 5120
_DIRBLKS = _N_PAD // _TC_ROWS
_TC_GRID = (_NC * _DIRBLKS,)

_DN = (((1,), (1,)), ((), ()))
_PREC = lax.Precision.DEFAULT


def _tc_body(s_ref, h_ref, deg_ref, W1, W2, mb, Wih, Whh, bih, bhh, out_ref):
    S = s_ref[...]
    hh = h_ref[...]
    dg = deg_ref[...]
    msg_self = lax.dot_general(hh, W2[0], _DN, precision=_PREC) + mb[0]
    aggr = lax.dot_general(S, W1[0], _DN, precision=_PREC) + dg * msg_self

    def mm(x, W, k):
        return lax.dot_general(x, W[0, k * HID:(k + 1) * HID], _DN, precision=_PREC)

    r = jax.nn.sigmoid(mm(aggr, Wih, 0) + bih[0, 0:1] + mm(hh, Whh, 0) + bhh[0, 0:1])
    z = jax.nn.sigmoid(mm(aggr, Wih, 1) + bih[0, 1:2] + mm(hh, Whh, 1) + bhh[0, 1:2])
    n = jnp.tanh(mm(aggr, Wih, 2) + bih[0, 2:3] + r * (mm(hh, Whh, 2) + bhh[0, 2:3]))
    out_ref[...] = (1.0 - z) * n + z * hh


def _rows(i):
    return (i, i * 0)


def _dirw(i):
    return (i // _DIRBLKS, i * 0, i * 0)


_W_SHAPES = [(NDIM, HID), (NDIM, HID), (1, NDIM), (3 * HID, NDIM), (3 * HID, HID), (3, HID), (3, HID)]

_TC_IN_SPECS = (
    [pl.BlockSpec((_TC_ROWS, HID), _rows),
     pl.BlockSpec((_TC_ROWS, HID), _rows),
     pl.BlockSpec((_TC_ROWS, 1), _rows)]
    + [pl.BlockSpec((1,) + shp, _dirw) for shp in _W_SHAPES]
)
_TC_OUT_SPECS = pl.BlockSpec((_TC_ROWS, HID), _rows)
_TC_OUT_SHAPE = jax.ShapeDtypeStruct((_NC * _N_PAD, HID), jnp.float32)


def _tc_layer(s, htab, deg, ws):
    return pl.pallas_call(
        _tc_body,
        grid=_TC_GRID,
        in_specs=_TC_IN_SPECS,
        out_specs=_TC_OUT_SPECS,
        out_shape=_TC_OUT_SHAPE,
    )(s, htab, deg, *ws)


def _prep_weights(fw, bw):
    def stack(f, b):
        return jnp.stack([f, b])
    f_msg_W, f_msg_b, f_Wih, f_Whh, f_bih, f_bhh = fw
    b_msg_W, b_msg_b, b_Wih, b_Whh, b_bih, b_bhh = bw
    return (
        stack(f_msg_W[:, :HID], b_msg_W[:, :HID]),
        stack(f_msg_W[:, HID:], b_msg_W[:, HID:]),
        stack(f_msg_b.reshape(1, NDIM), b_msg_b.reshape(1, NDIM)),
        stack(f_Wih, b_Wih),
        stack(f_Whh, b_Whh),
        stack(f_bih.reshape(3, HID), b_bih.reshape(3, HID)),
        stack(f_bhh.reshape(3, HID), b_bhh.reshape(3, HID)),
    )


def kernel(h, edge_index, fwd_msg_W, fwd_msg_b, fwd_Wih, fwd_Whh, fwd_bih, fwd_bhh,
           bwd_msg_W, bwd_msg_b, bwd_Wih, bwd_Whh, bwd_bih, bwd_bhh):
    h = h.astype(jnp.float32)
    src = edge_index[0].astype(jnp.int32)
    dst = edge_index[1].astype(jnp.int32)
    pad_idx = (jnp.arange(_E_PAD - N_EDGES, dtype=jnp.int32) % (_N_PAD - N_NODES)) + N_NODES
    srcp = jnp.concatenate([src, pad_idx]).reshape(_E_ROWS, _CHUNK)
    dstp = jnp.concatenate([dst, pad_idx]).reshape(_E_ROWS, _CHUNK)
    gidx = jnp.concatenate([srcp, dstp + jnp.int32(_N_PAD)], axis=0)
    sidx = jnp.concatenate([dstp, srcp], axis=0)
    htab = jnp.concatenate([
        jnp.pad(h[:, :HID], ((0, _N_PAD - N_NODES), (0, 0))),
        jnp.pad(h[:, HID:], ((0, _N_PAD - N_NODES), (0, 0))),
    ], axis=0)
    z2 = jnp.zeros((_ROWS_PER_TILE, HID), jnp.float32)
    z1 = jnp.zeros((_ROWS_PER_TILE,), jnp.float32)

    ws = [_prep_weights((fwd_msg_W[l], fwd_msg_b[l], fwd_Wih[l], fwd_Whh[l], fwd_bih[l], fwd_bhh[l]),
                        (bwd_msg_W[l], bwd_msg_b[l], bwd_Wih[l], bwd_Whh[l], bwd_bih[l], bwd_bhh[l]))
          for l in range(2)]

    res = _sc_segment_sums(htab, gidx, sidx, z2, z1, with_deg=True)
    deg_col = res[1].reshape(_NC * _N_PAD, 1)
    htab = _tc_layer(res[0], htab, deg_col, ws[0])
    res = _sc_segment_sums(htab, gidx, sidx, z2, z1, with_deg=False)
    htab = _tc_layer(res[0], htab, deg_col, ws[1])
    return jnp.concatenate([htab[:N_NODES], htab[_N_PAD:_N_PAD + N_NODES]], axis=1)


# speedup vs baseline: 20.0970x; 1.1132x over previous
import functools

import jax
import jax.numpy as jnp
from jax import lax
from jax.experimental import pallas as pl
from jax.experimental.pallas import tpu as pltpu
from jax.experimental.pallas import tpu_sc as plsc

NDIM = 128
HID = 64
N_NODES = 10000
N_EDGES = 320000

_NC, _NS, _LANES = 2, 16, 16
_N_PAD = 10240
_ROWS_PER_TILE = _N_PAD // _NS
_CHUNK = 256
_E_PAD = 327680
_E_ROWS = _E_PAD // _CHUNK
_CHUNKS_PER_T = _E_ROWS // _NS
_KROWS = 1
_STEPS = _CHUNKS_PER_T // _KROWS
_STEPS_HALF = _STEPS // 2


def _sc_segment_sums(htab, gidx, sidx, z2, z1, with_deg):
    mesh = plsc.VectorSubcoreMesh(
        core_axis_name="c", subcore_axis_name="s",
        num_cores=_NC, num_subcores=_NS)

    out_type = [jax.ShapeDtypeStruct((_NC * _N_PAD, HID), jnp.float32)]
    scratch = [
        pltpu.VMEM_SHARED((_N_PAD, HID), jnp.float32),
        pltpu.VMEM((_CHUNKS_PER_T, _CHUNK), jnp.int32),
        pltpu.VMEM((_CHUNKS_PER_T, _CHUNK), jnp.int32),
        pltpu.VMEM((2, _CHUNK, HID), jnp.float32),
        pltpu.SemaphoreType.DMA,
        pltpu.SemaphoreType.DMA,
    ]
    if with_deg:
        out_type += [jax.ShapeDtypeStruct((_NC * _N_PAD,), jnp.float32)]
        scratch += [
            pltpu.VMEM_SHARED((_N_PAD,), jnp.float32),
            pltpu.VMEM((_CHUNK,), jnp.float32),
        ]

    @functools.partial(
        pl.kernel,
        out_type=tuple(out_type),
        mesh=mesh,
        compiler_params=pltpu.CompilerParams(use_tc_tiling_on_sc=False),
        scratch_types=scratch,
    )
    def k(htab_hbm, gidx_hbm, sidx_hbm, z2_hbm, z1_hbm, *rest):
        if with_deg:
            (s_out, deg_out, acc_sh, gidx_v, sidx_v, rows, sem0, sem1,
             deg_sh, ones_v) = rest
        else:
            s_out, acc_sh, gidx_v, sidx_v, rows, sem0, sem1 = rest
        gsem = (sem0, sem1)
        c = lax.axis_index("c")
        s = lax.axis_index("s")
        r0 = s * _ROWS_PER_TILE
        pltpu.sync_copy(z2_hbm, acc_sh.at[pl.ds(r0, _ROWS_PER_TILE)])
        if with_deg:
            pltpu.sync_copy(z1_hbm, deg_sh.at[pl.ds(r0, _ROWS_PER_TILE)])
            for g in range(_CHUNK // _LANES):
                ones_v[pl.ds(g * _LANES, _LANES)] = jnp.full((_LANES,), 1.0, jnp.float32)
        slab0 = pl.multiple_of(c * jnp.int32(_E_ROWS) + s * jnp.int32(_CHUNKS_PER_T),
                               _CHUNKS_PER_T)
        pltpu.sync_copy(gidx_hbm.at[pl.ds(slab0, _CHUNKS_PER_T)], gidx_v)
        pltpu.sync_copy(sidx_hbm.at[pl.ds(slab0, _CHUNKS_PER_T)], sidx_v)
        plsc.subcore_barrier()

        last = jnp.int32(_STEPS - 1)

        def idx_rows(ref, p):
            return ref.at[p]

        def issue_gather(p, b):
            pc = jnp.minimum(p, last)
            pltpu.async_copy(htab_hbm.at[idx_rows(gidx_v, pc)], rows.at[jnp.int32(b)], gsem[b])

        def wait_gather(b):
            pltpu.make_async_copy(
                htab_hbm.at[idx_rows(gidx_v, jnp.int32(0))], rows.at[jnp.int32(b)], gsem[b]).wait()

        issue_gather(jnp.int32(0), 0)
        issue_gather(jnp.int32(1), 1)

        def body(j, carry):
            for b in range(2):
                p = j * jnp.int32(2) + jnp.int32(b)
                wait_gather(b)
                pltpu.sync_copy(rows.at[jnp.int32(b)], acc_sh.at[idx_rows(sidx_v, p)], add=True)
                if with_deg:
                    pltpu.sync_copy(ones_v, deg_sh.at[idx_rows(sidx_v, p)], add=True)
                issue_gather(p + jnp.int32(2), b)
            return carry

        lax.fori_loop(jnp.int32(0), jnp.int32(_STEPS_HALF), body, jnp.int32(0))
        wait_gather(0)
        wait_gather(1)
        plsc.subcore_barrier()

        out0 = pl.multiple_of(c * jnp.int32(_N_PAD) + r0, _ROWS_PER_TILE)
        pltpu.sync_copy(acc_sh.at[pl.ds(r0, _ROWS_PER_TILE)], s_out.at[pl.ds(out0, _ROWS_PER_TILE)])
        if with_deg:
            pltpu.sync_copy(deg_sh.at[pl.ds(r0, _ROWS_PER_TILE)], deg_out.at[pl.ds(out0, _ROWS_PER_TILE)])

    return k(htab, gidx, sidx, z2, z1)


_TC_ROWS = 5120
_DIRBLKS = _N_PAD // _TC_ROWS
_TC_GRID = (_NC * _DIRBLKS,)

_DN = (((1,), (1,)), ((), ()))
_PREC = lax.Precision.DEFAULT


def _tc_body(s_ref, h_ref, deg_ref, W1, W2, mb, Wih, Whh, bih, bhh, out_ref):
    S = s_ref[...]
    hh = h_ref[...]
    dg = deg_ref[...]
    msg_self = lax.dot_general(hh, W2[0], _DN, precision=_PREC) + mb[0]
    aggr = lax.dot_general(S, W1[0], _DN, precision=_PREC) + dg * msg_self

    def mm(x, W, k):
        return lax.dot_general(x, W[0, k * HID:(k + 1) * HID], _DN, precision=_PREC)

    r = jax.nn.sigmoid(mm(aggr, Wih, 0) + bih[0, 0:1] + mm(hh, Whh, 0) + bhh[0, 0:1])
    z = jax.nn.sigmoid(mm(aggr, Wih, 1) + bih[0, 1:2] + mm(hh, Whh, 1) + bhh[0, 1:2])
    n = jnp.tanh(mm(aggr, Wih, 2) + bih[0, 2:3] + r * (mm(hh, Whh, 2) + bhh[0, 2:3]))
    out_ref[...] = (1.0 - z) * n + z * hh


def _rows(i):
    return (i, i * 0)


def _dirw(i):
    return (i // _DIRBLKS, i * 0, i * 0)


_W_SHAPES = [(NDIM, HID), (NDIM, HID), (1, NDIM), (3 * HID, NDIM), (3 * HID, HID), (3, HID), (3, HID)]

_TC_IN_SPECS = (
    [pl.BlockSpec((_TC_ROWS, HID), _rows),
     pl.BlockSpec((_TC_ROWS, HID), _rows),
     pl.BlockSpec((_TC_ROWS, 1), _rows)]
    + [pl.BlockSpec((1,) + shp, _dirw) for shp in _W_SHAPES]
)
_TC_OUT_SPECS = pl.BlockSpec((_TC_ROWS, HID), _rows)
_TC_OUT_SHAPE = jax.ShapeDtypeStruct((_NC * _N_PAD, HID), jnp.float32)


def _tc_layer(s, htab, deg, ws):
    return pl.pallas_call(
        _tc_body,
        grid=_TC_GRID,
        in_specs=_TC_IN_SPECS,
        out_specs=_TC_OUT_SPECS,
        out_shape=_TC_OUT_SHAPE,
    )(s, htab, deg, *ws)


def _prep_weights(fw, bw):
    def stack(f, b):
        return jnp.stack([f, b])
    f_msg_W, f_msg_b, f_Wih, f_Whh, f_bih, f_bhh = fw
    b_msg_W, b_msg_b, b_Wih, b_Whh, b_bih, b_bhh = bw
    return (
        stack(f_msg_W[:, :HID], b_msg_W[:, :HID]),
        stack(f_msg_W[:, HID:], b_msg_W[:, HID:]),
        stack(f_msg_b.reshape(1, NDIM), b_msg_b.reshape(1, NDIM)),
        stack(f_Wih, b_Wih),
        stack(f_Whh, b_Whh),
        stack(f_bih.reshape(3, HID), b_bih.reshape(3, HID)),
        stack(f_bhh.reshape(3, HID), b_bhh.reshape(3, HID)),
    )


def kernel(h, edge_index, fwd_msg_W, fwd_msg_b, fwd_Wih, fwd_Whh, fwd_bih, fwd_bhh,
           bwd_msg_W, bwd_msg_b, bwd_Wih, bwd_Whh, bwd_bih, bwd_bhh):
    h = h.astype(jnp.float32)
    src = edge_index[0].astype(jnp.int32)
    dst = edge_index[1].astype(jnp.int32)
    pad_idx = (jnp.arange(_E_PAD - N_EDGES, dtype=jnp.int32) % (_N_PAD - N_NODES)) + N_NODES
    srcp = jnp.concatenate([src, pad_idx]).reshape(_E_ROWS, _CHUNK)
    dstp = jnp.concatenate([dst, pad_idx]).reshape(_E_ROWS, _CHUNK)
    gidx = jnp.concatenate([srcp, dstp + jnp.int32(_N_PAD)], axis=0)
    sidx = jnp.concatenate([dstp, srcp], axis=0)
    htab = jnp.concatenate([
        jnp.pad(h[:, :HID], ((0, _N_PAD - N_NODES), (0, 0))),
        jnp.pad(h[:, HID:], ((0, _N_PAD - N_NODES), (0, 0))),
    ], axis=0)
    z2 = jnp.zeros((_ROWS_PER_TILE, HID), jnp.float32)
    z1 = jnp.zeros((_ROWS_PER_TILE,), jnp.float32)

    ws = [_prep_weights((fwd_msg_W[l], fwd_msg_b[l], fwd_Wih[l], fwd_Whh[l], fwd_bih[l], fwd_bhh[l]),
                        (bwd_msg_W[l], bwd_msg_b[l], bwd_Wih[l], bwd_Whh[l], bwd_bih[l], bwd_bhh[l]))
          for l in range(2)]

    res = _sc_segment_sums(htab, gidx, sidx, z2, z1, with_deg=True)
    deg_col = res[1].reshape(_NC * _N_PAD, 1)
    htab = _tc_layer(res[0], htab, deg_col, ws[0])
    res = _sc_segment_sums(htab, gidx, sidx, z2, z1, with_deg=False)
    htab = _tc_layer(res[0], htab, deg_col, ws[1])
    return jnp.concatenate([htab[:N_NODES], htab[_N_PAD:_N_PAD + N_NODES]], axis=1)
